# baseline (speedup 1.0000x reference)
import functools

import jax
import jax.numpy as jnp
from jax import lax
from jax.experimental import pallas as pl
from jax.experimental.pallas import tpu as pltpu
from jax.experimental.pallas import tpu_sc as plsc

EMBED_DIM = 32
CHUNK = 128


@functools.lru_cache(maxsize=None)
def _build(B: int):
    info = plsc.get_sparse_core_info()
    NC, NS = info.num_cores, info.num_subcores
    NW = NC * NS
    b_per_w = B // NW
    n_chunks = b_per_w // CHUNK

    mesh = plsc.VectorSubcoreMesh(core_axis_name="c", subcore_axis_name="s")

    @functools.partial(
        pl.kernel,
        mesh=mesh,
        compiler_params=pltpu.CompilerParams(
            needs_layout_passes=False, use_tc_tiling_on_sc=False),
        out_type=jax.ShapeDtypeStruct((B,), jnp.float32),
        scratch_types=[
            pltpu.VMEM((n_chunks, CHUNK), jnp.int32),
            pltpu.VMEM((n_chunks, CHUNK), jnp.int32),
            pltpu.VMEM((b_per_w, EMBED_DIM), jnp.float32),
            pltpu.VMEM((b_per_w, EMBED_DIM), jnp.float32),
            pltpu.VMEM((EMBED_DIM,), jnp.float32),
            pltpu.VMEM((16,), jnp.float32),
            pltpu.VMEM((b_per_w,), jnp.float32),
            pltpu.VMEM((17 * 16,), jnp.float32),
            pltpu.SemaphoreType.DMA,
        ],
    )
    def kfn(uid_hbm, bid_hbm, utab_hbm, btab_hbm, w_hbm, b_hbm, out_hbm,
            uidx_v, bidx_v, urows_v, brows_v, w_v, b_v, out_v, tr_v, gsem):
        wid = lax.axis_index("s") * NC + lax.axis_index("c")
        base_row = wid * n_chunks

        pltpu.sync_copy(uid_hbm.at[pl.ds(base_row, n_chunks)], uidx_v)
        pltpu.sync_copy(bid_hbm.at[pl.ds(base_row, n_chunks)], bidx_v)
        pltpu.sync_copy(w_hbm, w_v)
        pltpu.sync_copy(b_hbm, b_v)

        copies = []
        for j in range(n_chunks):
            copies.append(pltpu.async_copy(
                utab_hbm.at[uidx_v.at[j]], urows_v.at[pl.ds(j * CHUNK, CHUNK)], gsem))
            copies.append(pltpu.async_copy(
                btab_hbm.at[bidx_v.at[j]], brows_v.at[pl.ds(j * CHUNK, CHUNK)], gsem))
        for c in copies:
            c.wait()

        w0 = w_v[pl.ds(0, 16)]
        w1 = w_v[pl.ds(16, 16)]
        fcb_vec = b_v[pl.ds(0, 16)]
        fcb = fcb_vec[0]
        col_base = lax.iota(jnp.int32, 16) * 17

        def group_body(g, carry):
            r0 = g * 16
            for r in range(16):
                u0 = urows_v[r0 + r, pl.ds(0, 16)]
                u1 = urows_v[r0 + r, pl.ds(16, 16)]
                bb0 = brows_v[r0 + r, pl.ds(0, 16)]
                bb1 = brows_v[r0 + r, pl.ds(16, 16)]
                p = u0 * bb0 * w0 + u1 * bb1 * w1
                plsc.store_scatter(tr_v, [col_base + r], p)
            acc = jnp.full((16,), fcb, dtype=jnp.float32)
            for d in range(16):
                acc = acc + tr_v[pl.ds(d * 17, 16)]
            out_v[pl.ds(r0, 16)] = acc
            return carry

        lax.fori_loop(0, b_per_w // 16, group_body, 0)

        pltpu.sync_copy(out_v, out_hbm.at[pl.ds(wid * b_per_w, b_per_w)])

    return kfn


def kernel(user_id, book_id, user_table, book_table, fc_w, fc_b):
    B = user_id.shape[0]
    uid2d = user_id.astype(jnp.int32).reshape(B // CHUNK, CHUNK)
    bid2d = book_id.astype(jnp.int32).reshape(B // CHUNK, CHUNK)
    w = fc_w.reshape(EMBED_DIM)
    b = jnp.pad(fc_b, (0, 15))
    return _build(B)(uid2d, bid2d, user_table, book_table, w, b)
